# Initial kernel scaffold; baseline (speedup 1.0000x reference)
import functools

import jax
import jax.numpy as jnp
from jax import lax
from jax.experimental import pallas as pl
from jax.experimental.pallas import tpu as pltpu
from jax.experimental.pallas import tpu_sc as plsc

N = 10000
E = 320000
D = 128
HID = 16
HEADS = 8

NC = 2
NS = 16
NW = NC * NS
L = 16

CH = 128
NCHUNK = (E + CH - 1) // CH
GPT = (NCHUNK + NW - 1) // NW
EPAD = GPT * NW * CH
ROWS_PER_TILE = N // NS
SRC_W = 16 + HEADS * HID
ACC2_W = 16

_STRIPES = ((0, 128), (128, 128), (256, 128), (384, 128), (512, 113))


def _lrelu(x):
    return jnp.maximum(x, 0.2 * x)


def _splat(v, h):
    return jnp.take(v, jnp.full((L,), h, jnp.int32), mode="promise_in_bounds")


def _tc1_body(x_ref, w1_ref, asrc_ref, adst_ref, tsrc_ref, tdst_ref, mrow_ref):
    h = jnp.dot(x_ref[...], w1_ref[...], preferred_element_type=jnp.float32)
    h3 = h.reshape(N, HEADS, HID)
    a_s = (h3 * asrc_ref[...][None]).sum(-1)
    a_d = (h3 * adst_ref[...][None]).sum(-1)
    m = _lrelu(jnp.max(a_s, axis=0) + jnp.max(a_d, axis=0))
    z8 = jnp.zeros((N, 8), jnp.float32)
    tsrc_ref[...] = jnp.concatenate([a_s, z8, h], axis=1)
    tdst_ref[...] = jnp.concatenate([a_d, z8], axis=1)
    mrow_ref[...] = jnp.concatenate([m, m])


def _tc1(x, W1, att_src1, att_dst1):
    return pl.pallas_call(
        _tc1_body,
        out_shape=(
            jax.ShapeDtypeStruct((N, SRC_W), jnp.float32),
            jax.ShapeDtypeStruct((N, 16), jnp.float32),
            jax.ShapeDtypeStruct((16,), jnp.float32),
        ),
    )(x, W1, att_src1, att_dst1)


def _sc_mesh():
    return plsc.VectorSubcoreMesh(
        core_axis_name="c", subcore_axis_name="s", num_cores=NC, num_subcores=NS
    )


def _zero_rows(buf, width):
    def zrow(i, carry):
        for j in range(width // L):
            buf[i, pl.ds(L * j, L)] = jnp.zeros((L,), jnp.float32)
        return carry

    lax.fori_loop(0, CH, zrow, 0)


def _sc1_body(tsrc_hbm, tdst_hbm, src_hbm, dst_hbm, mrow_hbm, out_hbm,
              acc, srcrows, dstrows, sidx, didx, mvec, gsem, gsem2):
    cid = lax.axis_index("c")
    sid = lax.axis_index("s")
    wid = sid * NC + cid

    _zero_rows(srcrows, SRC_W)
    base = sid * ROWS_PER_TILE
    for off, sz in _STRIPES:
        pltpu.sync_copy(srcrows.at[pl.ds(0, sz)], acc.at[pl.ds(base + off, sz)])
    plsc.subcore_barrier()

    pltpu.sync_copy(mrow_hbm, mvec)
    mw = mvec[...]
    mask8 = (lax.iota(jnp.int32, (L,)) < HEADS).astype(jnp.float32)

    def chunk_body(g, carry):
        q = g * NW + wid

        @pl.when(q < NCHUNK)
        def _():
            b = q * CH
            pltpu.sync_copy(src_hbm.at[pl.ds(b, CH)], sidx)
            pltpu.sync_copy(dst_hbm.at[pl.ds(b, CH)], didx)
            c1 = pltpu.async_copy(tsrc_hbm.at[sidx], srcrows, gsem)
            c2 = pltpu.async_copy(tdst_hbm.at[didx], dstrows, gsem2)
            c1.wait()
            c2.wait()

            def edge(e, ecarry):
                sr = srcrows[e, pl.ds(0, L)]
                dr = dstrows[e, pl.ds(0, L)]
                al = _lrelu(sr + dr)
                w = jnp.exp(al - mw) * mask8
                srcrows[e, pl.ds(0, L)] = w
                for h in range(HEADS):
                    wh = _splat(w, h)
                    o = 16 + HID * h
                    srcrows[e, pl.ds(o, L)] = srcrows[e, pl.ds(o, L)] * wh
                return ecarry

            lax.fori_loop(0, CH, edge, 0)
            pltpu.sync_copy(srcrows, acc.at[didx], add=True)

        return carry

    lax.fori_loop(0, GPT, chunk_body, 0)
    plsc.subcore_barrier()
    for off, sz in _STRIPES:
        pltpu.sync_copy(acc.at[pl.ds(base + off, sz)],
                        out_hbm.at[cid, pl.ds(base + off, sz)])


def _sc1(tsrc, tdst, src_p, dst_p, mrow):
    f = functools.partial(
        pl.kernel,
        out_type=jax.ShapeDtypeStruct((NC, N, SRC_W), jnp.float32),
        mesh=_sc_mesh(),
        scratch_types=[
            pltpu.VMEM_SHARED((N, SRC_W), jnp.float32),
            pltpu.VMEM((CH, SRC_W), jnp.float32),
            pltpu.VMEM((CH, 16), jnp.float32),
            pltpu.VMEM((CH,), jnp.int32),
            pltpu.VMEM((CH,), jnp.int32),
            pltpu.VMEM((L,), jnp.float32),
            pltpu.SemaphoreType.DMA,
            pltpu.SemaphoreType.DMA,
        ],
    )(_sc1_body)
    return f(tsrc, tdst, src_p, dst_p, mrow)


def _tc2_body(parts_ref, tsrc_ref, tdst_ref, mrow_ref, b1_ref, w2_ref,
              as2_ref, ad2_ref, tsrc2_ref, tdst2_ref, mrow2_ref):
    acc = parts_ref[0] + parts_ref[1]
    a_s = tsrc_ref[:, 0:8]
    h1 = tsrc_ref[:, 16:SRC_W]
    a_d = tdst_ref[:, 0:8]
    m1 = mrow_ref[pl.ds(0, 8)]
    wself = jnp.exp(_lrelu(a_s + a_d) - m1[None, :])
    wsum = acc[:, 0:8] + wself
    wself_r = jnp.broadcast_to(wself[:, :, None], (N, 8, HID)).reshape(N, D)
    wsum_r = jnp.broadcast_to(wsum[:, :, None], (N, 8, HID)).reshape(N, D)
    num = acc[:, 16:SRC_W] + wself_r * h1
    out1 = num / (wsum_r + 1e-16) + b1_ref[...][None, :]
    h2in = jnp.maximum(out1, 0.0)
    h2 = jnp.dot(h2in, w2_ref[...], preferred_element_type=jnp.float32)
    asv = as2_ref[...]
    adv = ad2_ref[...]
    as2 = h2[:, 0] * asv[0, 0] + h2[:, 1] * asv[0, 1]
    ad2 = h2[:, 0] * adv[0, 0] + h2[:, 1] * adv[0, 1]
    m2 = _lrelu(jnp.max(as2) + jnp.max(ad2))
    z13 = jnp.zeros((N, 13), jnp.float32)
    z15 = jnp.zeros((N, 15), jnp.float32)
    tsrc2_ref[...] = jnp.concatenate([as2[:, None], h2, z13], axis=1)
    tdst2_ref[...] = jnp.concatenate([ad2[:, None], z15], axis=1)
    mrow2_ref[...] = jnp.full((16,), m2, jnp.float32)


def _tc2(parts, tsrc, tdst, mrow, b1, W2, att_src2, att_dst2):
    return pl.pallas_call(
        _tc2_body,
        out_shape=(
            jax.ShapeDtypeStruct((N, 16), jnp.float32),
            jax.ShapeDtypeStruct((N, 16), jnp.float32),
            jax.ShapeDtypeStruct((16,), jnp.float32),
        ),
    )(parts, tsrc, tdst, mrow, b1, W2, att_src2, att_dst2)


def _sc2_body(tsrc2_hbm, tdst2_hbm, src_hbm, dst_hbm, mrow2_hbm, out_hbm,
              acc, srcrows, dstrows, sidx, didx, mvec, gsem, gsem2):
    cid = lax.axis_index("c")
    sid = lax.axis_index("s")
    wid = sid * NC + cid

    _zero_rows(srcrows, ACC2_W)
    base = sid * ROWS_PER_TILE
    for off, sz in _STRIPES:
        pltpu.sync_copy(srcrows.at[pl.ds(0, sz)], acc.at[pl.ds(base + off, sz)])
    plsc.subcore_barrier()

    pltpu.sync_copy(mrow2_hbm, mvec)
    m2 = mvec[...]
    lane = lax.iota(jnp.int32, (L,))
    zero = jnp.zeros((L,), jnp.float32)
    one = jnp.ones((L,), jnp.float32)

    def chunk_body(g, carry):
        q = g * NW + wid

        @pl.when(q < NCHUNK)
        def _():
            b = q * CH
            pltpu.sync_copy(src_hbm.at[pl.ds(b, CH)], sidx)
            pltpu.sync_copy(dst_hbm.at[pl.ds(b, CH)], didx)
            c1 = pltpu.async_copy(tsrc2_hbm.at[sidx], srcrows, gsem)
            c2 = pltpu.async_copy(tdst2_hbm.at[didx], dstrows, gsem2)
            c1.wait()
            c2.wait()

            def edge(e, ecarry):
                r = srcrows[e, pl.ds(0, L)]
                dd = dstrows[e, pl.ds(0, L)]
                v = r + dd
                a0 = _splat(v, 0)
                w = jnp.exp(_lrelu(a0) - m2)
                basis = jnp.where(lane == 0, one, jnp.where(lane < 3, r, zero))
                srcrows[e, pl.ds(0, L)] = w * basis
                return ecarry

            lax.fori_loop(0, CH, edge, 0)
            pltpu.sync_copy(srcrows, acc.at[didx], add=True)

        return carry

    lax.fori_loop(0, GPT, chunk_body, 0)
    plsc.subcore_barrier()
    for off, sz in _STRIPES:
        pltpu.sync_copy(acc.at[pl.ds(base + off, sz)],
                        out_hbm.at[cid, pl.ds(base + off, sz)])


def _sc2(tsrc2, tdst2, src_p, dst_p, mrow2):
    f = functools.partial(
        pl.kernel,
        out_type=jax.ShapeDtypeStruct((NC, N, ACC2_W), jnp.float32),
        mesh=_sc_mesh(),
        scratch_types=[
            pltpu.VMEM_SHARED((N, ACC2_W), jnp.float32),
            pltpu.VMEM((CH, ACC2_W), jnp.float32),
            pltpu.VMEM((CH, 16), jnp.float32),
            pltpu.VMEM((CH,), jnp.int32),
            pltpu.VMEM((CH,), jnp.int32),
            pltpu.VMEM((L,), jnp.float32),
            pltpu.SemaphoreType.DMA,
            pltpu.SemaphoreType.DMA,
        ],
    )(_sc2_body)
    return f(tsrc2, tdst2, src_p, dst_p, mrow2)


def _tc3_body(parts2_ref, tsrc2_ref, tdst2_ref, mrow2_ref, b2_ref, out_ref):
    acc2 = parts2_ref[0] + parts2_ref[1]
    as2 = tsrc2_ref[:, 0]
    h2 = tsrc2_ref[:, 1:3]
    ad2 = tdst2_ref[:, 0]
    m2 = mrow2_ref[0]
    wself = jnp.exp(_lrelu(as2 + ad2) - m2)
    wsum = acc2[:, 0] + wself
    num = acc2[:, 1:3] + wself[:, None] * h2
    out_ref[...] = num / (wsum[:, None] + 1e-16) + b2_ref[...][None, :]


def _tc3(parts2, tsrc2, tdst2, mrow2, b2):
    return pl.pallas_call(
        _tc3_body,
        out_shape=jax.ShapeDtypeStruct((N, 2), jnp.float32),
    )(parts2, tsrc2, tdst2, mrow2, b2)


def kernel(x, edge_index, W1, att_src1, att_dst1, b1, W2, att_src2, att_dst2, b2):
    src = edge_index[0]
    dst = edge_index[1]
    pad = EPAD - E
    zpad = jnp.zeros((pad,), jnp.int32)
    src_p = jnp.concatenate([src, zpad])
    dst_p = jnp.concatenate([dst, zpad])

    tsrc, tdst, mrow = _tc1(x, W1, att_src1, att_dst1)
    parts = _sc1(tsrc, tdst, src_p, dst_p, mrow)
    tsrc2, tdst2, mrow2 = _tc2(parts, tsrc, tdst, mrow, b1, W2, att_src2, att_dst2)
    parts2 = _sc2(tsrc2, tdst2, src_p, dst_p, mrow2)
    return _tc3(parts2, tsrc2, tdst2, mrow2, b2)


# speedup vs baseline: 54.3747x; 54.3747x over previous
import functools

import jax
import jax.numpy as jnp
from jax import lax
from jax.experimental import pallas as pl
from jax.experimental.pallas import tpu as pltpu
from jax.experimental.pallas import tpu_sc as plsc

N = 10000
E = 320000
D = 128
HID = 16
HEADS = 8

NC = 2
NS = 16
NW = NC * NS
L = 16

CH = 128
NCHUNK = (E + CH - 1) // CH
GPT = (NCHUNK + NW - 1) // NW
EPAD = GPT * NW * CH
N_PAD = 10112
ROWS_PER_TILE = N_PAD // NS
SRC_W = 16 + HEADS * HID
ACC2_W = 16

_STRIPES = ((0, 128), (128, 128), (256, 128), (384, 128), (512, 120))


_TC_PARAMS = pltpu.CompilerParams(vmem_limit_bytes=100 * 1024 * 1024)
_SC_PARAMS = pltpu.CompilerParams(use_tc_tiling_on_sc=False)


def _lrelu(x):
    return jnp.maximum(x, 0.2 * x)


_GATHER_DN = lax.GatherDimensionNumbers(
    offset_dims=(), collapsed_slice_dims=(0,), start_index_map=(0,))


def _splat(v, h):
    idx = jnp.full((L, 1), h, jnp.int32)
    return lax.gather(v, idx, _GATHER_DN, (1,),
                      mode=lax.GatherScatterMode.PROMISE_IN_BOUNDS)


def _tc1_body(x_ref, w1_ref, asrc_ref, adst_ref, tsrc_ref, tdst_ref, mrow_ref):
    h = jnp.dot(x_ref[...], w1_ref[...], preferred_element_type=jnp.float32)
    h3 = h.reshape(N, HEADS, HID)
    a_s = (h3 * asrc_ref[...][None]).sum(-1)
    a_d = (h3 * adst_ref[...][None]).sum(-1)
    m = _lrelu(jnp.max(a_s, axis=0) + jnp.max(a_d, axis=0))
    z8 = jnp.zeros((N, 8), jnp.float32)
    tsrc_ref[...] = jnp.concatenate([a_s, z8, h], axis=1)
    tdst_ref[...] = jnp.concatenate([a_d, z8], axis=1)
    mrow_ref[...] = jnp.concatenate([m, m])


def _tc1(x, W1, att_src1, att_dst1):
    return pl.pallas_call(
        _tc1_body,
        compiler_params=_TC_PARAMS,
        out_shape=(
            jax.ShapeDtypeStruct((N, SRC_W), jnp.float32),
            jax.ShapeDtypeStruct((N, 16), jnp.float32),
            jax.ShapeDtypeStruct((16,), jnp.float32),
        ),
    )(x, W1, att_src1, att_dst1)


def _sc_mesh():
    return plsc.VectorSubcoreMesh(
        core_axis_name="c", subcore_axis_name="s", num_cores=NC, num_subcores=NS
    )


def _zero_rows(buf, width):
    def zrow(i, carry):
        for j in range(width // L):
            buf[i, pl.ds(L * j, L)] = jnp.zeros((L,), jnp.float32)
        return carry

    lax.fori_loop(0, CH, zrow, 0)


def _sc1_body(tsrc_hbm, tdst_hbm, src_hbm, dst_hbm, mrow_hbm, out_hbm,
              acc, srcrows, dstrows, sidx, didx, mvec, gsem, gsem2):
    cid = lax.axis_index("c")
    sid = lax.axis_index("s")
    wid = sid * NC + cid

    _zero_rows(srcrows, SRC_W)
    base = sid * ROWS_PER_TILE
    for off, sz in _STRIPES:
        pltpu.sync_copy(srcrows.at[pl.ds(0, sz)], acc.at[pl.ds(base + off, sz)])
    plsc.subcore_barrier()

    pltpu.sync_copy(mrow_hbm, mvec)
    mw = mvec[...]
    zerov = jnp.zeros((L,), jnp.float32)

    def chunk_body(g, carry):
        q = g * NW + wid

        @pl.when(q < NCHUNK)
        def _():
            b = q * CH
            pltpu.sync_copy(src_hbm.at[pl.ds(b, CH)], sidx)
            pltpu.sync_copy(dst_hbm.at[pl.ds(b, CH)], didx)
            c1 = pltpu.async_copy(tsrc_hbm.at[sidx], srcrows, gsem)
            c2 = pltpu.async_copy(tdst_hbm.at[didx], dstrows, gsem2)
            c1.wait()
            c2.wait()

            def edge(e, ecarry):
                sr = srcrows[e, pl.ds(0, L)]
                dr = dstrows[e, pl.ds(0, L)]
                al = _lrelu(sr + dr)
                w = jnp.where(lax.iota(jnp.int32, L) < HEADS,
                              jnp.exp(al - mw), zerov)
                srcrows[e, pl.ds(0, L)] = w
                for h in range(HEADS):
                    wh = _splat(w, h)
                    o = 16 + HID * h
                    srcrows[e, pl.ds(o, L)] = srcrows[e, pl.ds(o, L)] * wh
                return ecarry

            lax.fori_loop(0, CH, edge, 0)
            pltpu.sync_copy(srcrows, acc.at[didx], add=True)

        return carry

    lax.fori_loop(0, GPT, chunk_body, 0)
    plsc.subcore_barrier()
    for off, sz in _STRIPES:
        pltpu.sync_copy(acc.at[pl.ds(base + off, sz)],
                        out_hbm.at[cid, pl.ds(base + off, sz)])


def _sc1(tsrc, tdst, src_p, dst_p, mrow):
    f = functools.partial(
        pl.kernel,
        out_type=jax.ShapeDtypeStruct((NC, N_PAD, SRC_W), jnp.float32),
        mesh=_sc_mesh(),
        compiler_params=_SC_PARAMS,
        scratch_types=[
            pltpu.VMEM_SHARED((N_PAD, SRC_W), jnp.float32),
            pltpu.VMEM((CH, SRC_W), jnp.float32),
            pltpu.VMEM((CH, 16), jnp.float32),
            pltpu.VMEM((CH,), jnp.int32),
            pltpu.VMEM((CH,), jnp.int32),
            pltpu.VMEM((L,), jnp.float32),
            pltpu.SemaphoreType.DMA,
            pltpu.SemaphoreType.DMA,
        ],
    )(_sc1_body)
    return f(tsrc, tdst, src_p, dst_p, mrow)


TC2_B = 2000


def _tc2_body(parts_ref, tsrc_ref, tdst_ref, mrow_ref, b1_ref, w2_ref,
              as2_ref, ad2_ref, tsrc2_ref, tdst2_ref, mrow2_ref):
    B = TC2_B
    step = pl.program_id(0)
    acc = parts_ref[0] + parts_ref[1]
    a_s = tsrc_ref[:, 0:8]
    h1 = tsrc_ref[:, 16:SRC_W]
    a_d = tdst_ref[:, 0:8]
    m1 = mrow_ref[pl.ds(0, 8)]
    wself = jnp.exp(_lrelu(a_s + a_d) - m1[None, :])
    wsum = acc[:, 0:8] + wself
    wself_r = jnp.broadcast_to(wself[:, :, None], (B, 8, HID)).reshape(B, D)
    wsum_r = jnp.broadcast_to(wsum[:, :, None], (B, 8, HID)).reshape(B, D)
    num = acc[:, 16:SRC_W] + wself_r * h1
    out1 = num / (wsum_r + 1e-16) + b1_ref[...][None, :]
    h2in = jnp.maximum(out1, 0.0)
    h2 = jnp.dot(h2in, w2_ref[...], preferred_element_type=jnp.float32)
    asv = as2_ref[...]
    adv = ad2_ref[...]
    as2 = h2[:, 0] * asv[0, 0] + h2[:, 1] * asv[0, 1]
    ad2 = h2[:, 0] * adv[0, 0] + h2[:, 1] * adv[0, 1]
    z13 = jnp.zeros((B, 13), jnp.float32)
    z15 = jnp.zeros((B, 15), jnp.float32)
    tsrc2_ref[...] = jnp.concatenate([as2[:, None], h2, z13], axis=1)
    tdst2_ref[...] = jnp.concatenate([ad2[:, None], z15], axis=1)
    cur = jnp.concatenate([jnp.full((8,), jnp.max(as2), jnp.float32),
                           jnp.full((8,), jnp.max(ad2), jnp.float32)])

    @pl.when(step == 0)
    def _():
        mrow2_ref[...] = cur

    @pl.when(step > 0)
    def _():
        mrow2_ref[...] = jnp.maximum(mrow2_ref[...], cur)


def _tc2(parts, tsrc, tdst, mrow, b1, W2, att_src2, att_dst2):
    B = TC2_B
    grid = N // B
    return pl.pallas_call(
        _tc2_body,
        compiler_params=_TC_PARAMS,
        grid=(grid,),
        in_specs=[
            pl.BlockSpec((2, B, SRC_W), lambda i: (0, i, 0)),
            pl.BlockSpec((B, SRC_W), lambda i: (i, 0)),
            pl.BlockSpec((B, 16), lambda i: (i, 0)),
            pl.BlockSpec((16,), lambda i: (0,)),
            pl.BlockSpec((D,), lambda i: (0,)),
            pl.BlockSpec((D, 2), lambda i: (0, 0)),
            pl.BlockSpec((1, 2), lambda i: (0, 0)),
            pl.BlockSpec((1, 2), lambda i: (0, 0)),
        ],
        out_specs=(
            pl.BlockSpec((B, 16), lambda i: (i, 0)),
            pl.BlockSpec((B, 16), lambda i: (i, 0)),
            pl.BlockSpec((16,), lambda i: (0,)),
        ),
        out_shape=(
            jax.ShapeDtypeStruct((N, 16), jnp.float32),
            jax.ShapeDtypeStruct((N, 16), jnp.float32),
            jax.ShapeDtypeStruct((16,), jnp.float32),
        ),
    )(parts, tsrc, tdst, mrow, b1, W2, att_src2, att_dst2)


def _sc2_body(tsrc2_hbm, tdst2_hbm, src_hbm, dst_hbm, mrow2_hbm, out_hbm,
              acc, srcrows, dstrows, sidx, didx, mvec, gsem, gsem2):
    cid = lax.axis_index("c")
    sid = lax.axis_index("s")
    wid = sid * NC + cid

    _zero_rows(srcrows, ACC2_W)
    base = sid * ROWS_PER_TILE
    for off, sz in _STRIPES:
        pltpu.sync_copy(srcrows.at[pl.ds(0, sz)], acc.at[pl.ds(base + off, sz)])
    plsc.subcore_barrier()

    pltpu.sync_copy(mrow2_hbm, mvec)
    mraw = mvec[...]
    m2 = _lrelu(_splat(mraw, 0) + _splat(mraw, 8))
    lane = lax.iota(jnp.int32, L)
    zero = jnp.zeros((L,), jnp.float32)
    one = jnp.ones((L,), jnp.float32)

    def chunk_body(g, carry):
        q = g * NW + wid

        @pl.when(q < NCHUNK)
        def _():
            b = q * CH
            pltpu.sync_copy(src_hbm.at[pl.ds(b, CH)], sidx)
            pltpu.sync_copy(dst_hbm.at[pl.ds(b, CH)], didx)
            c1 = pltpu.async_copy(tsrc2_hbm.at[sidx], srcrows, gsem)
            c2 = pltpu.async_copy(tdst2_hbm.at[didx], dstrows, gsem2)
            c1.wait()
            c2.wait()

            def edge(e, ecarry):
                r = srcrows[e, pl.ds(0, L)]
                dd = dstrows[e, pl.ds(0, L)]
                v = r + dd
                a0 = _splat(v, 0)
                w = jnp.exp(_lrelu(a0) - m2)
                basis = jnp.where(lane == 0, one, jnp.where(lane < 3, r, zero))
                srcrows[e, pl.ds(0, L)] = w * basis
                return ecarry

            lax.fori_loop(0, CH, edge, 0)
            pltpu.sync_copy(srcrows, acc.at[didx], add=True)

        return carry

    lax.fori_loop(0, GPT, chunk_body, 0)
    plsc.subcore_barrier()
    for off, sz in _STRIPES:
        pltpu.sync_copy(acc.at[pl.ds(base + off, sz)],
                        out_hbm.at[cid, pl.ds(base + off, sz)])


def _sc2(tsrc2, tdst2, src_p, dst_p, mrow2):
    f = functools.partial(
        pl.kernel,
        out_type=jax.ShapeDtypeStruct((NC, N_PAD, ACC2_W), jnp.float32),
        mesh=_sc_mesh(),
        compiler_params=_SC_PARAMS,
        scratch_types=[
            pltpu.VMEM_SHARED((N_PAD, ACC2_W), jnp.float32),
            pltpu.VMEM((CH, ACC2_W), jnp.float32),
            pltpu.VMEM((CH, 16), jnp.float32),
            pltpu.VMEM((CH,), jnp.int32),
            pltpu.VMEM((CH,), jnp.int32),
            pltpu.VMEM((L,), jnp.float32),
            pltpu.SemaphoreType.DMA,
            pltpu.SemaphoreType.DMA,
        ],
    )(_sc2_body)
    return f(tsrc2, tdst2, src_p, dst_p, mrow2)


def _tc3_body(parts2_ref, tsrc2_ref, tdst2_ref, mrow2_ref, b2_ref, out_ref):
    acc2 = parts2_ref[0, pl.ds(0, N)] + parts2_ref[1, pl.ds(0, N)]
    as2 = tsrc2_ref[:, 0]
    h2 = tsrc2_ref[:, 1:3]
    ad2 = tdst2_ref[:, 0]
    m2 = _lrelu(mrow2_ref[0] + mrow2_ref[8])
    wself = jnp.exp(_lrelu(as2 + ad2) - m2)
    wsum = acc2[:, 0] + wself
    num = acc2[:, 1:3] + wself[:, None] * h2
    out_ref[...] = num / (wsum[:, None] + 1e-16) + b2_ref[...][None, :]


def _tc3(parts2, tsrc2, tdst2, mrow2, b2):
    return pl.pallas_call(
        _tc3_body,
        compiler_params=_TC_PARAMS,
        out_shape=jax.ShapeDtypeStruct((N, 2), jnp.float32),
    )(parts2, tsrc2, tdst2, mrow2, b2)


def kernel(x, edge_index, W1, att_src1, att_dst1, b1, W2, att_src2, att_dst2, b2):
    src = edge_index[0]
    dst = edge_index[1]
    pad = EPAD - E
    zpad = jnp.zeros((pad,), jnp.int32)
    src_p = jnp.concatenate([src, zpad])
    dst_p = jnp.concatenate([dst, zpad])

    tsrc, tdst, mrow = _tc1(x, W1, att_src1, att_dst1)
    parts = _sc1(tsrc, tdst, src_p, dst_p, mrow)
    tsrc2, tdst2, mrow2 = _tc2(parts, tsrc, tdst, mrow, b1, W2, att_src2, att_dst2)
    parts2 = _sc2(tsrc2, tdst2, src_p, dst_p, mrow2)
    return _tc3(parts2, tsrc2, tdst2, mrow2, b2)


# speedup vs baseline: 67.1118x; 1.2342x over previous
import functools

import jax
import jax.numpy as jnp
from jax import lax
from jax.experimental import pallas as pl
from jax.experimental.pallas import tpu as pltpu
from jax.experimental.pallas import tpu_sc as plsc

N = 10000
E = 320000
D = 128
HID = 16
HEADS = 8

NC = 2
NS = 16
NW = NC * NS
L = 16

CH = 128
NCHUNK = (E + CH - 1) // CH
GPT = (NCHUNK + NW - 1) // NW
EPAD = GPT * NW * CH
N_PAD = 10112
ROWS_PER_TILE = N_PAD // NS
SRC_W = 16 + HEADS * HID
ACC2_W = 16

_STRIPES = ((0, 128), (128, 128), (256, 128), (384, 128), (512, 120))


_TC_PARAMS = pltpu.CompilerParams(vmem_limit_bytes=100 * 1024 * 1024)
_SC_PARAMS = pltpu.CompilerParams(use_tc_tiling_on_sc=False)


def _lrelu(x):
    return jnp.maximum(x, 0.2 * x)


_GATHER_DN = lax.GatherDimensionNumbers(
    offset_dims=(), collapsed_slice_dims=(0,), start_index_map=(0,))


def _splat(v, h):
    idx = jnp.full((L, 1), h, jnp.int32)
    return lax.gather(v, idx, _GATHER_DN, (1,),
                      mode=lax.GatherScatterMode.PROMISE_IN_BOUNDS)


def _tc1_body(x_ref, w1_ref, asrc_ref, adst_ref, tsrc_ref, tdst_ref, mrow_ref):
    h = jnp.dot(x_ref[...], w1_ref[...], preferred_element_type=jnp.float32)
    h3 = h.reshape(N, HEADS, HID)
    a_s = (h3 * asrc_ref[...][None]).sum(-1)
    a_d = (h3 * adst_ref[...][None]).sum(-1)
    m = _lrelu(jnp.max(a_s, axis=0) + jnp.max(a_d, axis=0))
    z8 = jnp.zeros((N, 8), jnp.float32)
    tsrc_ref[...] = jnp.concatenate([a_s, z8, h], axis=1)
    tdst_ref[...] = jnp.concatenate([a_d, z8], axis=1)
    mrow_ref[...] = jnp.concatenate([m, m])


def _tc1(x, W1, att_src1, att_dst1):
    return pl.pallas_call(
        _tc1_body,
        compiler_params=_TC_PARAMS,
        out_shape=(
            jax.ShapeDtypeStruct((N, SRC_W), jnp.float32),
            jax.ShapeDtypeStruct((N, 16), jnp.float32),
            jax.ShapeDtypeStruct((16,), jnp.float32),
        ),
    )(x, W1, att_src1, att_dst1)


def _sc_mesh():
    return plsc.VectorSubcoreMesh(
        core_axis_name="c", subcore_axis_name="s", num_cores=NC, num_subcores=NS
    )


def _zero_rows(buf, width):
    def zrow(i, carry):
        for j in range(width // L):
            buf[i, pl.ds(L * j, L)] = jnp.zeros((L,), jnp.float32)
        return carry

    lax.fori_loop(0, CH, zrow, 0)


def _sc1_body(tsrc_hbm, tdst_hbm, src_hbm, dst_hbm, mrow_hbm, out_hbm,
              acc, srcrows, dstrows, sidx, didx, mvec, gsem, gsem2):
    cid = lax.axis_index("c")
    sid = lax.axis_index("s")
    wid = sid * NC + cid

    _zero_rows(srcrows, SRC_W)
    base = sid * ROWS_PER_TILE
    for off, sz in _STRIPES:
        pltpu.sync_copy(srcrows.at[pl.ds(0, sz)], acc.at[pl.ds(base + off, sz)])
    plsc.subcore_barrier()

    pltpu.sync_copy(mrow_hbm, mvec)
    mw = mvec[...]
    zerov = jnp.zeros((L,), jnp.float32)

    def chunk_body(g, carry):
        q = g * NW + wid

        @pl.when(q < NCHUNK)
        def _():
            b = q * CH
            pltpu.sync_copy(src_hbm.at[pl.ds(b, CH)], sidx)
            pltpu.sync_copy(dst_hbm.at[pl.ds(b, CH)], didx)
            c1 = pltpu.async_copy(tsrc_hbm.at[sidx], srcrows, gsem)
            c2 = pltpu.async_copy(tdst_hbm.at[didx], dstrows, gsem2)
            c1.wait()
            c2.wait()

            def edge(e4, ecarry):
                ws = []
                for k in range(4):
                    e = e4 * 4 + k
                    sr = srcrows[e, pl.ds(0, L)]
                    dr = dstrows[e, pl.ds(0, L)]
                    al = _lrelu(sr + dr)
                    w = jnp.where(lax.iota(jnp.int32, L) < HEADS,
                                  jnp.exp(al - mw), zerov)
                    srcrows[e, pl.ds(0, L)] = w
                    ws.append(w)
                for k in range(4):
                    e = e4 * 4 + k
                    for h in range(HEADS):
                        wh = _splat(ws[k], h)
                        o = 16 + HID * h
                        srcrows[e, pl.ds(o, L)] = srcrows[e, pl.ds(o, L)] * wh
                return ecarry

            lax.fori_loop(0, CH // 4, edge, 0)
            pltpu.sync_copy(srcrows, acc.at[didx], add=True)

        return carry

    lax.fori_loop(0, GPT, chunk_body, 0)
    plsc.subcore_barrier()
    for off, sz in _STRIPES:
        pltpu.sync_copy(acc.at[pl.ds(base + off, sz)],
                        out_hbm.at[cid, pl.ds(base + off, sz)])


def _sc1(tsrc, tdst, src_p, dst_p, mrow):
    f = functools.partial(
        pl.kernel,
        out_type=jax.ShapeDtypeStruct((NC, N_PAD, SRC_W), jnp.float32),
        mesh=_sc_mesh(),
        compiler_params=_SC_PARAMS,
        scratch_types=[
            pltpu.VMEM_SHARED((N_PAD, SRC_W), jnp.float32),
            pltpu.VMEM((CH, SRC_W), jnp.float32),
            pltpu.VMEM((CH, 16), jnp.float32),
            pltpu.VMEM((CH,), jnp.int32),
            pltpu.VMEM((CH,), jnp.int32),
            pltpu.VMEM((L,), jnp.float32),
            pltpu.SemaphoreType.DMA,
            pltpu.SemaphoreType.DMA,
        ],
    )(_sc1_body)
    return f(tsrc, tdst, src_p, dst_p, mrow)


TC2_B = 2000


def _tc2_body(parts_ref, tsrc_ref, tdst_ref, mrow_ref, b1_ref, w2_ref,
              as2_ref, ad2_ref, tsrc2_ref, tdst2_ref, mrow2_ref):
    B = TC2_B
    step = pl.program_id(0)
    acc = parts_ref[0] + parts_ref[1]
    a_s = tsrc_ref[:, 0:8]
    h1 = tsrc_ref[:, 16:SRC_W]
    a_d = tdst_ref[:, 0:8]
    m1 = mrow_ref[pl.ds(0, 8)]
    wself = jnp.exp(_lrelu(a_s + a_d) - m1[None, :])
    wsum = acc[:, 0:8] + wself
    wself_r = jnp.broadcast_to(wself[:, :, None], (B, 8, HID)).reshape(B, D)
    wsum_r = jnp.broadcast_to(wsum[:, :, None], (B, 8, HID)).reshape(B, D)
    num = acc[:, 16:SRC_W] + wself_r * h1
    out1 = num / (wsum_r + 1e-16) + b1_ref[...][None, :]
    h2in = jnp.maximum(out1, 0.0)
    h2 = jnp.dot(h2in, w2_ref[...], preferred_element_type=jnp.float32)
    asv = as2_ref[...]
    adv = ad2_ref[...]
    as2 = h2[:, 0] * asv[0, 0] + h2[:, 1] * asv[0, 1]
    ad2 = h2[:, 0] * adv[0, 0] + h2[:, 1] * adv[0, 1]
    z13 = jnp.zeros((B, 13), jnp.float32)
    z15 = jnp.zeros((B, 15), jnp.float32)
    tsrc2_ref[...] = jnp.concatenate([as2[:, None], h2, z13], axis=1)
    tdst2_ref[...] = jnp.concatenate([ad2[:, None], z15], axis=1)
    cur = jnp.concatenate([jnp.full((8,), jnp.max(as2), jnp.float32),
                           jnp.full((8,), jnp.max(ad2), jnp.float32)])

    @pl.when(step == 0)
    def _():
        mrow2_ref[...] = cur

    @pl.when(step > 0)
    def _():
        mrow2_ref[...] = jnp.maximum(mrow2_ref[...], cur)


def _tc2(parts, tsrc, tdst, mrow, b1, W2, att_src2, att_dst2):
    B = TC2_B
    grid = N // B
    return pl.pallas_call(
        _tc2_body,
        compiler_params=_TC_PARAMS,
        grid=(grid,),
        in_specs=[
            pl.BlockSpec((2, B, SRC_W), lambda i: (0, i, 0)),
            pl.BlockSpec((B, SRC_W), lambda i: (i, 0)),
            pl.BlockSpec((B, 16), lambda i: (i, 0)),
            pl.BlockSpec((16,), lambda i: (0,)),
            pl.BlockSpec((D,), lambda i: (0,)),
            pl.BlockSpec((D, 2), lambda i: (0, 0)),
            pl.BlockSpec((1, 2), lambda i: (0, 0)),
            pl.BlockSpec((1, 2), lambda i: (0, 0)),
        ],
        out_specs=(
            pl.BlockSpec((B, 16), lambda i: (i, 0)),
            pl.BlockSpec((B, 16), lambda i: (i, 0)),
            pl.BlockSpec((16,), lambda i: (0,)),
        ),
        out_shape=(
            jax.ShapeDtypeStruct((N, 16), jnp.float32),
            jax.ShapeDtypeStruct((N, 16), jnp.float32),
            jax.ShapeDtypeStruct((16,), jnp.float32),
        ),
    )(parts, tsrc, tdst, mrow, b1, W2, att_src2, att_dst2)


def _sc2_body(tsrc2_hbm, tdst2_hbm, src_hbm, dst_hbm, mrow2_hbm, out_hbm,
              acc, srcrows, dstrows, sidx, didx, mvec, gsem, gsem2):
    cid = lax.axis_index("c")
    sid = lax.axis_index("s")
    wid = sid * NC + cid

    _zero_rows(srcrows, ACC2_W)
    base = sid * ROWS_PER_TILE
    for off, sz in _STRIPES:
        pltpu.sync_copy(srcrows.at[pl.ds(0, sz)], acc.at[pl.ds(base + off, sz)])
    plsc.subcore_barrier()

    pltpu.sync_copy(mrow2_hbm, mvec)
    mraw = mvec[...]
    m2 = _lrelu(_splat(mraw, 0) + _splat(mraw, 8))
    lane = lax.iota(jnp.int32, L)
    zero = jnp.zeros((L,), jnp.float32)
    one = jnp.ones((L,), jnp.float32)

    def chunk_body(g, carry):
        q = g * NW + wid

        @pl.when(q < NCHUNK)
        def _():
            b = q * CH
            pltpu.sync_copy(src_hbm.at[pl.ds(b, CH)], sidx)
            pltpu.sync_copy(dst_hbm.at[pl.ds(b, CH)], didx)
            c1 = pltpu.async_copy(tsrc2_hbm.at[sidx], srcrows, gsem)
            c2 = pltpu.async_copy(tdst2_hbm.at[didx], dstrows, gsem2)
            c1.wait()
            c2.wait()

            def edge(e8, ecarry):
                for k in range(8):
                    e = e8 * 8 + k
                    r = srcrows[e, pl.ds(0, L)]
                    dd = dstrows[e, pl.ds(0, L)]
                    v = r + dd
                    a0 = _splat(v, 0)
                    w = jnp.exp(_lrelu(a0) - m2)
                    basis = jnp.where(lane == 0, one,
                                      jnp.where(lane < 3, r, zero))
                    srcrows[e, pl.ds(0, L)] = w * basis
                return ecarry

            lax.fori_loop(0, CH // 8, edge, 0)
            pltpu.sync_copy(srcrows, acc.at[didx], add=True)

        return carry

    lax.fori_loop(0, GPT, chunk_body, 0)
    plsc.subcore_barrier()
    for off, sz in _STRIPES:
        pltpu.sync_copy(acc.at[pl.ds(base + off, sz)],
                        out_hbm.at[cid, pl.ds(base + off, sz)])


def _sc2(tsrc2, tdst2, src_p, dst_p, mrow2):
    f = functools.partial(
        pl.kernel,
        out_type=jax.ShapeDtypeStruct((NC, N_PAD, ACC2_W), jnp.float32),
        mesh=_sc_mesh(),
        compiler_params=_SC_PARAMS,
        scratch_types=[
            pltpu.VMEM_SHARED((N_PAD, ACC2_W), jnp.float32),
            pltpu.VMEM((CH, ACC2_W), jnp.float32),
            pltpu.VMEM((CH, 16), jnp.float32),
            pltpu.VMEM((CH,), jnp.int32),
            pltpu.VMEM((CH,), jnp.int32),
            pltpu.VMEM((L,), jnp.float32),
            pltpu.SemaphoreType.DMA,
            pltpu.SemaphoreType.DMA,
        ],
    )(_sc2_body)
    return f(tsrc2, tdst2, src_p, dst_p, mrow2)


def _tc3_body(parts2_ref, tsrc2_ref, tdst2_ref, mrow2_ref, b2_ref, out_ref):
    acc2 = parts2_ref[0, pl.ds(0, N)] + parts2_ref[1, pl.ds(0, N)]
    as2 = tsrc2_ref[:, 0]
    h2 = tsrc2_ref[:, 1:3]
    ad2 = tdst2_ref[:, 0]
    m2 = _lrelu(mrow2_ref[0] + mrow2_ref[8])
    wself = jnp.exp(_lrelu(as2 + ad2) - m2)
    wsum = acc2[:, 0] + wself
    num = acc2[:, 1:3] + wself[:, None] * h2
    out_ref[...] = num / (wsum[:, None] + 1e-16) + b2_ref[...][None, :]


def _tc3(parts2, tsrc2, tdst2, mrow2, b2):
    return pl.pallas_call(
        _tc3_body,
        compiler_params=_TC_PARAMS,
        out_shape=jax.ShapeDtypeStruct((N, 2), jnp.float32),
    )(parts2, tsrc2, tdst2, mrow2, b2)


def kernel(x, edge_index, W1, att_src1, att_dst1, b1, W2, att_src2, att_dst2, b2):
    src = edge_index[0]
    dst = edge_index[1]
    pad = EPAD - E
    zpad = jnp.zeros((pad,), jnp.int32)
    src_p = jnp.concatenate([src, zpad])
    dst_p = jnp.concatenate([dst, zpad])

    tsrc, tdst, mrow = _tc1(x, W1, att_src1, att_dst1)
    parts = _sc1(tsrc, tdst, src_p, dst_p, mrow)
    tsrc2, tdst2, mrow2 = _tc2(parts, tsrc, tdst, mrow, b1, W2, att_src2, att_dst2)
    parts2 = _sc2(tsrc2, tdst2, src_p, dst_p, mrow2)
    return _tc3(parts2, tsrc2, tdst2, mrow2, b2)


# speedup vs baseline: 67.1340x; 1.0003x over previous
import functools

import jax
import jax.numpy as jnp
from jax import lax
from jax.experimental import pallas as pl
from jax.experimental.pallas import tpu as pltpu
from jax.experimental.pallas import tpu_sc as plsc

N = 10000
E = 320000
D = 128
HID = 16
HEADS = 8

NC = 2
NS = 16
NW = NC * NS
L = 16

CH = 128
NCHUNK = (E + CH - 1) // CH
GPT = (NCHUNK + NW - 1) // NW
EPAD = GPT * NW * CH
N_ACC = 10112
ROWS_PER_TILE = N_ACC // NS
SRC_W = 16 + HEADS * HID
ACC2_W = 16

_STRIPES = ((0, 128), (128, 128), (256, 128), (384, 128), (512, 120))


_TC_PARAMS = pltpu.CompilerParams(vmem_limit_bytes=100 * 1024 * 1024)
_SC_PARAMS = pltpu.CompilerParams(use_tc_tiling_on_sc=False)


def _lrelu(x):
    return jnp.maximum(x, 0.2 * x)


_GATHER_DN = lax.GatherDimensionNumbers(
    offset_dims=(), collapsed_slice_dims=(0,), start_index_map=(0,))


def _splat(v, h):
    idx = jnp.full((L, 1), h, jnp.int32)
    return lax.gather(v, idx, _GATHER_DN, (1,),
                      mode=lax.GatherScatterMode.PROMISE_IN_BOUNDS)


def _tc1_body(x_ref, w1_ref, asrc_ref, adst_ref, tsrc_ref, tdst_ref, mrow_ref):
    h = jnp.dot(x_ref[...], w1_ref[...], preferred_element_type=jnp.float32)
    h3 = h.reshape(N, HEADS, HID)
    a_s = (h3 * asrc_ref[...][None]).sum(-1)
    a_d = (h3 * adst_ref[...][None]).sum(-1)
    m = _lrelu(jnp.max(a_s, axis=0) + jnp.max(a_d, axis=0))
    z8 = jnp.zeros((N, 8), jnp.float32)
    tsrc_ref[...] = jnp.concatenate([a_s, z8, h], axis=1)
    tdst_ref[...] = jnp.concatenate([a_d, z8], axis=1)
    mrow_ref[...] = jnp.concatenate([m, m])


def _tc1(x, W1, att_src1, att_dst1):
    return pl.pallas_call(
        _tc1_body,
        compiler_params=_TC_PARAMS,
        out_shape=(
            jax.ShapeDtypeStruct((N, SRC_W), jnp.float32),
            jax.ShapeDtypeStruct((N, 16), jnp.float32),
            jax.ShapeDtypeStruct((16,), jnp.float32),
        ),
    )(x, W1, att_src1, att_dst1)


def _sc_mesh():
    return plsc.VectorSubcoreMesh(
        core_axis_name="c", subcore_axis_name="s", num_cores=NC, num_subcores=NS
    )


def _zero_rows(buf, width):
    def zrow(i, carry):
        for j in range(width // L):
            buf[i, pl.ds(L * j, L)] = jnp.zeros((L,), jnp.float32)
        return carry

    lax.fori_loop(0, CH, zrow, 0)


def _sc1_body(tsrc_hbm, tdst_hbm, src_hbm, dst_hbm, mrow_hbm, out_hbm,
              acc, srcrows0, dstrows0, sidx0, didx0, mvec, gsem0, hsem0):
    cid = lax.axis_index("c")
    sid = lax.axis_index("s")
    wid = sid * NC + cid
    slots = ((srcrows0, dstrows0, sidx0, didx0, gsem0, hsem0),)

    _zero_rows(srcrows0, SRC_W)
    base = sid * ROWS_PER_TILE
    for off, sz in _STRIPES:
        pltpu.sync_copy(srcrows0.at[pl.ds(0, sz)], acc.at[pl.ds(base + off, sz)])
    plsc.subcore_barrier()

    pltpu.sync_copy(mrow_hbm, mvec)
    mw = mvec[...]
    zerov = jnp.zeros((L,), jnp.float32)

    def valid(g):
        return g * NW + wid < NCHUNK

    def issue_gather(g, sl):
        sr, dr, si, di, gs, hs = sl
        b = (g * NW + wid) * CH
        pltpu.sync_copy(src_hbm.at[pl.ds(b, CH)], si)
        pltpu.sync_copy(dst_hbm.at[pl.ds(b, CH)], di)
        c1 = pltpu.async_copy(tsrc_hbm.at[si], sr, gs)
        c2 = pltpu.async_copy(tdst_hbm.at[di], dr, hs)
        return c1, c2

    def compute(sl):
        sr_ref, dr_ref, si, di, gs, hs = sl

        def edge(e4, ecarry):
            ws = []
            for k in range(4):
                e = e4 * 4 + k
                srv = sr_ref[e, pl.ds(0, L)]
                drv = dr_ref[e, pl.ds(0, L)]
                al = _lrelu(srv + drv)
                w = jnp.where(lax.iota(jnp.int32, L) < HEADS,
                              jnp.exp(al - mw), zerov)
                sr_ref[e, pl.ds(0, L)] = w
                ws.append(w)
            for k in range(4):
                e = e4 * 4 + k
                for h in range(HEADS):
                    wh = _splat(ws[k], h)
                    o = 16 + HID * h
                    sr_ref[e, pl.ds(o, L)] = sr_ref[e, pl.ds(o, L)] * wh
            return ecarry

        lax.fori_loop(0, CH // 4, edge, 0)

    def scatter(sl):
        sr, dr, si, di, gs, hs = sl
        pltpu.sync_copy(sr, acc.at[di], add=True)

    def chunk_body(g, carry):
        @pl.when(valid(g))
        def _():
            c1, c2 = issue_gather(g, slots[0])
            c1.wait()
            c2.wait()
            compute(slots[0])
            scatter(slots[0])
        return carry

    lax.fori_loop(0, GPT, chunk_body, 0)
    plsc.subcore_barrier()
    for off, sz in _STRIPES:
        pltpu.sync_copy(acc.at[pl.ds(base + off, sz)],
                        out_hbm.at[cid, pl.ds(base + off, sz)])


def _sc1(tsrc, tdst, src_p, dst_p, mrow):
    f = functools.partial(
        pl.kernel,
        out_type=jax.ShapeDtypeStruct((NC, N_ACC, SRC_W), jnp.float32),
        mesh=_sc_mesh(),
        compiler_params=_SC_PARAMS,
        scratch_types=[
            pltpu.VMEM_SHARED((N_ACC, SRC_W), jnp.float32),
            pltpu.VMEM((CH, SRC_W), jnp.float32),
            pltpu.VMEM((CH, 16), jnp.float32),
            pltpu.VMEM((CH,), jnp.int32),
            pltpu.VMEM((CH,), jnp.int32),
            pltpu.VMEM((L,), jnp.float32),
            pltpu.SemaphoreType.DMA,
            pltpu.SemaphoreType.DMA,
        ],
    )(_sc1_body)
    return f(tsrc, tdst, src_p, dst_p, mrow)


TC2_B = 2000


def _tc2_body(parts_ref, tsrc_ref, tdst_ref, mrow_ref, b1_ref, w2_ref,
              as2_ref, ad2_ref, tsrc2_ref, tdst2_ref, mrow2_ref):
    B = TC2_B
    step = pl.program_id(0)
    acc = parts_ref[0] + parts_ref[1]
    a_s = tsrc_ref[:, 0:8]
    h1 = tsrc_ref[:, 16:SRC_W]
    a_d = tdst_ref[:, 0:8]
    m1 = mrow_ref[pl.ds(0, 8)]
    wself = jnp.exp(_lrelu(a_s + a_d) - m1[None, :])
    wsum = acc[:, 0:8] + wself
    wself_r = jnp.broadcast_to(wself[:, :, None], (B, 8, HID)).reshape(B, D)
    wsum_r = jnp.broadcast_to(wsum[:, :, None], (B, 8, HID)).reshape(B, D)
    num = acc[:, 16:SRC_W] + wself_r * h1
    out1 = num / (wsum_r + 1e-16) + b1_ref[...][None, :]
    h2in = jnp.maximum(out1, 0.0)
    h2 = jnp.dot(h2in, w2_ref[...], preferred_element_type=jnp.float32)
    asv = as2_ref[...]
    adv = ad2_ref[...]
    as2 = h2[:, 0] * asv[0, 0] + h2[:, 1] * asv[0, 1]
    ad2 = h2[:, 0] * adv[0, 0] + h2[:, 1] * adv[0, 1]
    z13 = jnp.zeros((B, 13), jnp.float32)
    z15 = jnp.zeros((B, 15), jnp.float32)
    tsrc2_ref[...] = jnp.concatenate([as2[:, None], h2, z13], axis=1)
    tdst2_ref[...] = jnp.concatenate([ad2[:, None], z15], axis=1)
    cur = jnp.concatenate([jnp.full((8,), jnp.max(as2), jnp.float32),
                           jnp.full((8,), jnp.max(ad2), jnp.float32)])

    @pl.when(step == 0)
    def _():
        mrow2_ref[...] = cur

    @pl.when(step > 0)
    def _():
        mrow2_ref[...] = jnp.maximum(mrow2_ref[...], cur)


def _tc2(parts, tsrc, tdst, mrow, b1, W2, att_src2, att_dst2):
    B = TC2_B
    grid = N // B
    return pl.pallas_call(
        _tc2_body,
        compiler_params=_TC_PARAMS,
        grid=(grid,),
        in_specs=[
            pl.BlockSpec((2, B, SRC_W), lambda i: (0, i, 0)),
            pl.BlockSpec((B, SRC_W), lambda i: (i, 0)),
            pl.BlockSpec((B, 16), lambda i: (i, 0)),
            pl.BlockSpec((16,), lambda i: (0,)),
            pl.BlockSpec((D,), lambda i: (0,)),
            pl.BlockSpec((D, 2), lambda i: (0, 0)),
            pl.BlockSpec((1, 2), lambda i: (0, 0)),
            pl.BlockSpec((1, 2), lambda i: (0, 0)),
        ],
        out_specs=(
            pl.BlockSpec((B, 16), lambda i: (i, 0)),
            pl.BlockSpec((B, 16), lambda i: (i, 0)),
            pl.BlockSpec((16,), lambda i: (0,)),
        ),
        out_shape=(
            jax.ShapeDtypeStruct((N, 16), jnp.float32),
            jax.ShapeDtypeStruct((N, 16), jnp.float32),
            jax.ShapeDtypeStruct((16,), jnp.float32),
        ),
    )(parts, tsrc, tdst, mrow, b1, W2, att_src2, att_dst2)


def _sc2_body(tsrc2_hbm, tdst2_hbm, src_hbm, dst_hbm, mrow2_hbm, out_hbm,
              acc, srcrows, dstrows, sidx, didx, mvec, gsem, gsem2):
    cid = lax.axis_index("c")
    sid = lax.axis_index("s")
    wid = sid * NC + cid

    _zero_rows(srcrows, ACC2_W)
    base = sid * ROWS_PER_TILE
    for off, sz in _STRIPES:
        pltpu.sync_copy(srcrows.at[pl.ds(0, sz)], acc.at[pl.ds(base + off, sz)])
    plsc.subcore_barrier()

    pltpu.sync_copy(mrow2_hbm, mvec)
    mraw = mvec[...]
    m2 = _lrelu(_splat(mraw, 0) + _splat(mraw, 8))
    lane = lax.iota(jnp.int32, L)
    zero = jnp.zeros((L,), jnp.float32)
    one = jnp.ones((L,), jnp.float32)

    def chunk_body(g, carry):
        q = g * NW + wid

        @pl.when(q < NCHUNK)
        def _():
            b = q * CH
            pltpu.sync_copy(src_hbm.at[pl.ds(b, CH)], sidx)
            pltpu.sync_copy(dst_hbm.at[pl.ds(b, CH)], didx)
            c1 = pltpu.async_copy(tsrc2_hbm.at[sidx], srcrows, gsem)
            c2 = pltpu.async_copy(tdst2_hbm.at[didx], dstrows, gsem2)
            c1.wait()
            c2.wait()

            def edge(e8, ecarry):
                for k in range(8):
                    e = e8 * 8 + k
                    r = srcrows[e, pl.ds(0, L)]
                    dd = dstrows[e, pl.ds(0, L)]
                    v = r + dd
                    a0 = _splat(v, 0)
                    w = jnp.exp(_lrelu(a0) - m2)
                    basis = jnp.where(lane == 0, one,
                                      jnp.where(lane < 3, r, zero))
                    srcrows[e, pl.ds(0, L)] = w * basis
                return ecarry

            lax.fori_loop(0, CH // 8, edge, 0)
            pltpu.sync_copy(srcrows, acc.at[didx], add=True)

        return carry

    lax.fori_loop(0, GPT, chunk_body, 0)
    plsc.subcore_barrier()
    for off, sz in _STRIPES:
        pltpu.sync_copy(acc.at[pl.ds(base + off, sz)],
                        out_hbm.at[cid, pl.ds(base + off, sz)])


def _sc2(tsrc2, tdst2, src_p, dst_p, mrow2):
    f = functools.partial(
        pl.kernel,
        out_type=jax.ShapeDtypeStruct((NC, N_ACC, ACC2_W), jnp.float32),
        mesh=_sc_mesh(),
        compiler_params=_SC_PARAMS,
        scratch_types=[
            pltpu.VMEM_SHARED((N_ACC, ACC2_W), jnp.float32),
            pltpu.VMEM((CH, ACC2_W), jnp.float32),
            pltpu.VMEM((CH, 16), jnp.float32),
            pltpu.VMEM((CH,), jnp.int32),
            pltpu.VMEM((CH,), jnp.int32),
            pltpu.VMEM((L,), jnp.float32),
            pltpu.SemaphoreType.DMA,
            pltpu.SemaphoreType.DMA,
        ],
    )(_sc2_body)
    return f(tsrc2, tdst2, src_p, dst_p, mrow2)


def _tc3_body(parts2_ref, tsrc2_ref, tdst2_ref, mrow2_ref, b2_ref, out_ref):
    acc2 = parts2_ref[0, pl.ds(0, N)] + parts2_ref[1, pl.ds(0, N)]
    as2 = tsrc2_ref[:, 0]
    h2 = tsrc2_ref[:, 1:3]
    ad2 = tdst2_ref[:, 0]
    m2 = _lrelu(mrow2_ref[0] + mrow2_ref[8])
    wself = jnp.exp(_lrelu(as2 + ad2) - m2)
    wsum = acc2[:, 0] + wself
    num = acc2[:, 1:3] + wself[:, None] * h2
    out_ref[...] = num / (wsum[:, None] + 1e-16) + b2_ref[...][None, :]


def _tc3(parts2, tsrc2, tdst2, mrow2, b2):
    return pl.pallas_call(
        _tc3_body,
        compiler_params=_TC_PARAMS,
        out_shape=jax.ShapeDtypeStruct((N, 2), jnp.float32),
    )(parts2, tsrc2, tdst2, mrow2, b2)


def kernel(x, edge_index, W1, att_src1, att_dst1, b1, W2, att_src2, att_dst2, b2):
    src = edge_index[0]
    dst = edge_index[1]
    pad = EPAD - E
    src_p = jnp.concatenate([src, jnp.zeros((pad,), jnp.int32)])
    dst_p = jnp.concatenate([dst, jnp.full((pad,), N, jnp.int32)])

    tsrc, tdst, mrow = _tc1(x, W1, att_src1, att_dst1)
    parts = _sc1(tsrc, tdst, src_p, dst_p, mrow)
    tsrc2, tdst2, mrow2 = _tc2(parts, tsrc, tdst, mrow, b1, W2, att_src2, att_dst2)
    parts2 = _sc2(tsrc2, tdst2, src_p, dst_p, mrow2)
    return _tc3(parts2, tsrc2, tdst2, mrow2, b2)
